# Initial kernel scaffold; baseline (speedup 1.0000x reference)
import jax
import jax.numpy as jnp
from jax.experimental import pallas as pl


def kernel(ray, grid, ray_min, ray_max):
    raise NotImplementedError("write your pallas kernel here")


# speedup vs baseline: 3.0351x; 3.0351x over previous
import functools

import jax
import jax.numpy as jnp
from jax import lax
from jax.experimental import pallas as pl
from jax.experimental.pallas import tpu as pltpu
from jax.experimental.pallas import tpu_sc as plsc

C = 16
D0 = D1 = D2 = 160
P = D0 * D1 * D2
S0 = D1 * D2
S1 = D2
N_RAYS = 1048576

NC, NS, L = 2, 16, 16
NW = NC * NS


def _make_sc_kernel(n_rays, k):
    rpw = n_rays // NW
    nch = rpw // k
    kh = k // 2
    g_half = kh // L
    nq = 2 * 8

    mesh = plsc.VectorSubcoreMesh(
        core_axis_name="c", subcore_axis_name="s",
        num_cores=NC, num_subcores=NS)

    @functools.partial(
        pl.kernel,
        out_type=jax.ShapeDtypeStruct((n_rays, C), jnp.float32),
        mesh=mesh,
        scratch_types=[
            pltpu.VMEM((3 * k,), jnp.float32),
            pltpu.VMEM((nq, 1, kh), jnp.int32),
            pltpu.VMEM((nq * kh,), jnp.float32),
            pltpu.VMEM((8 * k, C), jnp.float32),
            pltpu.VMEM((k, C), jnp.float32),
            pltpu.VMEM((8 * L,), jnp.float32),
            pltpu.SemaphoreType.DMA,
        ],
        compiler_params=pltpu.CompilerParams(
            use_tc_tiling_on_sc=False, needs_layout_passes=False),
    )
    def sc_kernel(rayt, tab, prm, out, rv, idxv, wv, rowsv, outv, prmv, sem):
        wid = lax.axis_index("s") * NC + lax.axis_index("c")
        base = wid * rpw
        pltpu.sync_copy(prm, prmv)
        iot = lax.iota(jnp.int32, L)
        csplat = [jnp.full((L,), c, jnp.int32) for c in range(C)]

        @pl.loop(0, nch)
        def _chunk(ci):
            off = base + ci * k
            pltpu.sync_copy(rayt.at[pl.ds(off, k)], rv.at[pl.ds(0, k)])
            pltpu.sync_copy(rayt.at[pl.ds(n_rays + off, k)],
                            rv.at[pl.ds(k, k)])
            pltpu.sync_copy(rayt.at[pl.ds(2 * n_rays + off, k)],
                            rv.at[pl.ds(2 * k, k)])

            sc0 = prmv[pl.ds(0 * L, L)]
            sc1 = prmv[pl.ds(1 * L, L)]
            sc2 = prmv[pl.ds(2 * L, L)]
            of0 = prmv[pl.ds(3 * L, L)]
            of1 = prmv[pl.ds(4 * L, L)]
            of2 = prmv[pl.ds(5 * L, L)]

            for h in range(2):
                @pl.loop(0, g_half)
                def _grp(gg, h=h):
                    r = h * kh + gg * L
                    x0 = rv[pl.ds(0 * k + r, L)]
                    x1 = rv[pl.ds(1 * k + r, L)]
                    x2 = rv[pl.ds(2 * k + r, L)]
                    p0 = x0 * sc0 + of0
                    p1 = x1 * sc1 + of1
                    p2 = x2 * sc2 + of2
                    b0 = p0.astype(jnp.int32)
                    b1 = p1.astype(jnp.int32)
                    b2 = p2.astype(jnp.int32)
                    w0 = p0 - b0.astype(jnp.float32)
                    w1 = p1 - b1.astype(jnp.float32)
                    w2 = p2 - b2.astype(jnp.float32)
                    o0, o1, o2 = 1.0 - w0, 1.0 - w1, 1.0 - w2
                    lin = b0 * S0 + b1 * S1 + b2
                    c00, c01 = o0 * o1, o0 * w1
                    c10, c11 = w0 * o1, w0 * w1
                    lins = (lin, lin + 1, lin + S1, lin + S1 + 1,
                            lin + S0, lin + S0 + 1,
                            lin + S0 + S1, lin + S0 + S1 + 1)
                    wts = (c00 * o2, c00 * w2, c01 * o2, c01 * w2,
                           c10 * o2, c10 * w2, c11 * o2, c11 * w2)
                    d = pl.ds(gg * L, L)
                    for j in range(8):
                        q = 2 * j + h
                        idxv[q, 0, d] = lins[j]
                        wv[pl.ds(q * kh + gg * L, L)] = wts[j]

            cps = []
            for q in range(nq):
                j, h = q // 2, q % 2
                dst = rowsv.at[pl.ds(j * k + h * kh, kh)]
                cps.append(pltpu.async_copy(tab.at[idxv.at[q, 0]], dst, sem))
            for cp in cps:
                cp.wait()

            for h in range(2):
                @pl.loop(0, g_half)
                def _cmb(gg, h=h):
                    rbase = h * kh + gg * L
                    ridx = rbase + iot
                    accs = [jnp.zeros((L,), jnp.float32) for _ in range(C)]
                    for j in range(8):
                        q = 2 * j + h
                        wj = wv[pl.ds(q * kh + gg * L, L)]
                        rows_idx = (j * k + rbase) + iot
                        for c in range(C):
                            v = plsc.load_gather(rowsv, [rows_idx, csplat[c]])
                            accs[c] = accs[c] + wj * v
                    for c in range(C):
                        plsc.store_scatter(outv, [ridx, csplat[c]], accs[c])

            pltpu.sync_copy(outv, out.at[pl.ds(off, k)])

    return sc_kernel


_sc_kernel = None


def kernel(ray, grid, ray_min, ray_max):
    global _sc_kernel
    if _sc_kernel is None:
        _sc_kernel = _make_sc_kernel(N_RAYS, 256)
    tab = grid.reshape(C, P).T
    rayt = ray.T.reshape(3 * ray.shape[0])
    sizes = jnp.array([D0 - 1, D1 - 1, D2 - 1], jnp.float32)
    scale = sizes / (ray_max - ray_min)
    offs = -ray_min * scale
    prm = jnp.concatenate(
        [jnp.repeat(scale, L), jnp.repeat(offs, L),
         jnp.zeros((2 * L,), jnp.float32)])
    return _sc_kernel(rayt, tab, prm)
